# baseline (speedup 1.0000x reference)
import functools

import jax
import jax.numpy as jnp
from jax import lax
from jax.experimental import pallas as pl
from jax.experimental.pallas import tpu as pltpu
from jax.experimental.pallas import tpu_sc as plsc

N_CODES = 8192
EMBED_DIM = 256
ROWS = 9216
ROW_BLOCK = 512
CODE_CHUNK = 512
N_ROW_BLOCKS = ROWS // ROW_BLOCK
N_CODE_CHUNKS = N_CODES // CODE_CHUNK
BIG_F = 3.0e38
BIG_I = 2**30


def _argmin_body(x_ref, xb_ref, embt_ref, embtb_ref, enc_ref, loss_ref,
                 perp_ref, counts_acc, loss_acc):
    i = pl.program_id(0)
    x = x_ref[...]
    xb = xb_ref[...]
    x2 = jnp.sum(x * x, axis=1, keepdims=True)

    best = jnp.full((ROW_BLOCK, 1), BIG_F, dtype=jnp.float32)
    bestidx = jnp.zeros((ROW_BLOCK, 1), dtype=jnp.int32)
    col_iota = lax.broadcasted_iota(jnp.int32, (ROW_BLOCK, CODE_CHUNK), 1)

    for c in range(N_CODE_CHUNKS):
        e = embt_ref[:, c * CODE_CHUNK:(c + 1) * CODE_CHUNK]
        eb = embtb_ref[:, c * CODE_CHUNK:(c + 1) * CODE_CHUNK]
        dot = lax.dot_general(
            xb, eb, (((1,), (0,)), ((), ())),
            preferred_element_type=jnp.float32)
        e2 = jnp.sum(e * e, axis=0, keepdims=True)
        s = jnp.sqrt(jnp.maximum((x2 - 2.0 * dot) + e2, 0.0))
        m = jnp.min(s, axis=1, keepdims=True)
        idxc = jnp.min(jnp.where(s == m, col_iota, BIG_I), axis=1, keepdims=True)
        upd = m < best
        best = jnp.where(upd, m, best)
        bestidx = jnp.where(upd, idxc + c * CODE_CHUNK, bestidx)

    enc_ref[...] = jnp.broadcast_to(bestidx, (ROW_BLOCK, 8))

    block_loss = jnp.sum(best * best)

    @pl.when(i == 0)
    def _():
        loss_acc[0, 0] = 0.0
        counts_acc[...] = jnp.zeros((1, N_CODES), jnp.float32)

    loss_acc[0, 0] += block_loss

    for c in range(N_CODE_CHUNKS):
        ids = lax.broadcasted_iota(jnp.int32, (1, CODE_CHUNK), 1) + c * CODE_CHUNK
        eq = (bestidx == ids).astype(jnp.float32)
        counts_acc[:, c * CODE_CHUNK:(c + 1) * CODE_CHUNK] += jnp.sum(
            eq, axis=0, keepdims=True)

    @pl.when(i == N_ROW_BLOCKS - 1)
    def _():
        loss_ref[0, 0] = loss_acc[0, 0] * (0.25 / (ROWS * EMBED_DIM))
        p = counts_acc[...] * (1.0 / ROWS)
        ent = jnp.sum(p * jnp.log(p + 1e-10))
        perp_ref[0, 0] = jnp.exp(-ent)


def _distance_argmin(flat, flatb, embt, embtb):
    return pl.pallas_call(
        _argmin_body,
        grid=(N_ROW_BLOCKS,),
        in_specs=[
            pl.BlockSpec((ROW_BLOCK, EMBED_DIM), lambda i: (i, 0)),
            pl.BlockSpec((ROW_BLOCK, EMBED_DIM), lambda i: (i, 0)),
            pl.BlockSpec((EMBED_DIM, N_CODES), lambda i: (0, 0)),
            pl.BlockSpec((EMBED_DIM, N_CODES), lambda i: (0, 0)),
        ],
        out_specs=[
            pl.BlockSpec((ROW_BLOCK, 8), lambda i: (i, 0)),
            pl.BlockSpec(memory_space=pltpu.SMEM, block_shape=(1, 1),
                         index_map=lambda i: (0, 0)),
            pl.BlockSpec(memory_space=pltpu.SMEM, block_shape=(1, 1),
                         index_map=lambda i: (0, 0)),
        ],
        out_shape=[
            jax.ShapeDtypeStruct((ROWS, 8), jnp.int32),
            jax.ShapeDtypeStruct((1, 1), jnp.float32),
            jax.ShapeDtypeStruct((1, 1), jnp.float32),
        ],
        scratch_shapes=[
            pltpu.VMEM((1, N_CODES), jnp.float32),
            pltpu.SMEM((1, 1), jnp.float32),
        ],
    )(flat, flatb, embt, embtb)


def _sc_gather(table, idx):
    info = plsc.get_sparse_core_info()
    nw = info.num_cores * info.num_subcores
    b_per_w = ROWS // nw
    mesh = plsc.VectorSubcoreMesh(core_axis_name="c", subcore_axis_name="s")

    @functools.partial(
        pl.kernel, mesh=mesh,
        out_type=jax.ShapeDtypeStruct((ROWS, EMBED_DIM), jnp.float32),
        scratch_types=[
            pltpu.VMEM((b_per_w,), jnp.int32),
            pltpu.VMEM((b_per_w, EMBED_DIM), jnp.float32),
            pltpu.SemaphoreType.DMA,
        ],
    )
    def k(table_hbm, idx_hbm, out_hbm, idx_v, rows_v, sem):
        wid = lax.axis_index("s") * info.num_cores + lax.axis_index("c")
        base = wid * b_per_w
        pltpu.sync_copy(idx_hbm.at[pl.ds(base, b_per_w)], idx_v)
        pltpu.async_copy(table_hbm.at[idx_v], rows_v, sem).wait()
        pltpu.sync_copy(rows_v, out_hbm.at[pl.ds(base, b_per_w)])

    return k(table, idx)


def kernel(z, embeddings):
    B = z.shape[0]
    spatial = z.shape[2:]
    flat = jnp.moveaxis(z, 1, -1).reshape(-1, EMBED_DIM)
    embt = embeddings.T

    enc8, loss, perp = _distance_argmin(
        flat, flat.astype(jnp.bfloat16), embt, embt.astype(jnp.bfloat16))
    enc_flat = enc8[:, 0]

    gathered = _sc_gather(embeddings, enc_flat)

    emb = jnp.moveaxis(gathered.reshape((B,) + spatial + (EMBED_DIM,)), -1, 1)
    enc = enc_flat.reshape((B,) + spatial)
    return (emb, enc, loss.reshape(()), perp.reshape(()))


# speedup vs baseline: 1.4440x; 1.4440x over previous
import functools

import jax
import jax.numpy as jnp
from jax import lax
from jax.experimental import pallas as pl
from jax.experimental.pallas import tpu as pltpu
from jax.experimental.pallas import tpu_sc as plsc

N_CODES = 8192
EMBED_DIM = 256
ROWS = 9216
ROW_BLOCK = 512
CODE_CHUNK = 512
N_ROW_BLOCKS = ROWS // ROW_BLOCK
N_CODE_CHUNKS = N_CODES // CODE_CHUNK
BIG_F = 3.0e38


def _tree_argmin0(v, ii):
    h = v.shape[0]
    while h > 1:
        h //= 2
        a, b = v[:h], v[h:]
        ia, ib = ii[:h], ii[h:]
        take_b = b < a
        v = jnp.where(take_b, b, a)
        ii = jnp.where(take_b, ib, ia)
    return v, ii


def _argmin_body(xt_ref, emb_ref, enc_ref, loss_ref, ebs, e2s, loss_acc):
    i = pl.program_id(0)
    xt = xt_ref[...]
    xbm = (xt * -2.0).astype(jnp.bfloat16)
    x2 = jnp.sum(xt * xt, axis=0, keepdims=True)

    @pl.when(i == 0)
    def _():
        loss_acc[0, 0] = 0.0
        for c in range(N_CODE_CHUNKS):
            e = emb_ref[c * CODE_CHUNK:(c + 1) * CODE_CHUNK, :]
            ebs[c * CODE_CHUNK:(c + 1) * CODE_CHUNK, :] = e.astype(jnp.bfloat16)
            e2s[c * CODE_CHUNK:(c + 1) * CODE_CHUNK, :] = jnp.sum(
                e * e, axis=1, keepdims=True)

    best = jnp.full((1, ROW_BLOCK), BIG_F, dtype=jnp.float32)
    bestidx = jnp.zeros((1, ROW_BLOCK), dtype=jnp.int32)
    row_iota = lax.broadcasted_iota(jnp.int32, (CODE_CHUNK, ROW_BLOCK), 0)

    for c in range(N_CODE_CHUNKS):
        eb = ebs[c * CODE_CHUNK:(c + 1) * CODE_CHUNK, :]
        e2 = e2s[c * CODE_CHUNK:(c + 1) * CODE_CHUNK, :]
        dotm = lax.dot_general(
            eb, xbm, (((1,), (0,)), ((), ())),
            preferred_element_type=jnp.float32)
        s = jnp.sqrt(jnp.maximum((x2 + dotm) + e2, 0.0))
        m, idx = _tree_argmin0(s, row_iota + c * CODE_CHUNK)
        upd = m < best
        best = jnp.where(upd, m, best)
        bestidx = jnp.where(upd, idx, bestidx)

    enc_ref[...] = bestidx[None]

    loss_acc[0, 0] += jnp.sum(best * best)

    @pl.when(i == N_ROW_BLOCKS - 1)
    def _():
        loss_ref[0, 0] = loss_acc[0, 0] * (0.25 / (ROWS * EMBED_DIM))


def _distance_argmin(flatt, emb):
    return pl.pallas_call(
        _argmin_body,
        grid=(N_ROW_BLOCKS,),
        in_specs=[
            pl.BlockSpec((EMBED_DIM, ROW_BLOCK), lambda i: (0, i)),
            pl.BlockSpec((N_CODES, EMBED_DIM), lambda i: (0, 0)),
        ],
        out_specs=[
            pl.BlockSpec((1, 1, ROW_BLOCK), lambda i: (i, 0, 0)),
            pl.BlockSpec(memory_space=pltpu.SMEM, block_shape=(1, 1),
                         index_map=lambda i: (0, 0)),
        ],
        out_shape=[
            jax.ShapeDtypeStruct((N_ROW_BLOCKS, 1, ROW_BLOCK), jnp.int32),
            jax.ShapeDtypeStruct((1, 1), jnp.float32),
        ],
        scratch_shapes=[
            pltpu.VMEM((N_CODES, EMBED_DIM), jnp.bfloat16),
            pltpu.VMEM((N_CODES, 1), jnp.float32),
            pltpu.SMEM((1, 1), jnp.float32),
        ],
    )(flatt, emb)


def _hist_body(enc_ref, perp_ref, counts):
    enc = enc_ref[...]
    for c in range(N_CODE_CHUNKS):
        ids = lax.broadcasted_iota(jnp.int32, (CODE_CHUNK, 1), 0) + c * CODE_CHUNK
        acc = jnp.zeros((CODE_CHUNK, 1), jnp.float32)
        for r in range(N_ROW_BLOCKS):
            eq = (enc[r] == ids).astype(jnp.bfloat16)
            acc = acc + lax.dot_general(
                eq, jnp.ones((ROW_BLOCK, 1), jnp.bfloat16),
                (((1,), (0,)), ((), ())),
                preferred_element_type=jnp.float32)
        counts[c * CODE_CHUNK:(c + 1) * CODE_CHUNK, :] = acc
    p = counts[...] * (1.0 / ROWS)
    ent = jnp.sum(p * jnp.log(p + 1e-10))
    perp_ref[0, 0] = jnp.exp(-ent)


def _histogram_perplexity(enc2d):
    return pl.pallas_call(
        _hist_body,
        out_specs=pl.BlockSpec(memory_space=pltpu.SMEM),
        out_shape=jax.ShapeDtypeStruct((1, 1), jnp.float32),
        scratch_shapes=[pltpu.VMEM((N_CODES, 1), jnp.float32)],
    )(enc2d)


def _sc_gather(table, idx):
    info = plsc.get_sparse_core_info()
    nw = info.num_cores * info.num_subcores
    b_per_w = ROWS // nw
    mesh = plsc.VectorSubcoreMesh(core_axis_name="c", subcore_axis_name="s")

    @functools.partial(
        pl.kernel, mesh=mesh,
        out_type=jax.ShapeDtypeStruct((ROWS, EMBED_DIM), jnp.float32),
        scratch_types=[
            pltpu.VMEM((b_per_w,), jnp.int32),
            pltpu.VMEM((b_per_w, EMBED_DIM), jnp.float32),
            pltpu.SemaphoreType.DMA,
        ],
    )
    def k(table_hbm, idx_hbm, out_hbm, idx_v, rows_v, sem):
        wid = lax.axis_index("s") * info.num_cores + lax.axis_index("c")
        base = wid * b_per_w
        pltpu.sync_copy(idx_hbm.at[pl.ds(base, b_per_w)], idx_v)
        pltpu.async_copy(table_hbm.at[idx_v], rows_v, sem).wait()
        pltpu.sync_copy(rows_v, out_hbm.at[pl.ds(base, b_per_w)])

    return k(table, idx)


def kernel(z, embeddings):
    B = z.shape[0]
    spatial = z.shape[2:]
    flatt = jnp.moveaxis(z, 1, 0).reshape(EMBED_DIM, -1)

    enc2d, loss = _distance_argmin(flatt, embeddings)
    enc_flat = enc2d.reshape(-1)

    perp = _histogram_perplexity(enc2d)
    gathered = _sc_gather(embeddings, enc_flat)

    emb = jnp.moveaxis(gathered.reshape((B,) + spatial + (EMBED_DIM,)), -1, 1)
    enc = enc_flat.reshape((B,) + spatial)
    return (emb, enc, loss.reshape(()), perp.reshape(()))
